# baseline (speedup 1.0000x reference)
import jax
import jax.numpy as jnp
from jax.experimental import pallas as pl
from jax.experimental.pallas import tpu as pltpu

TOKENS = 2048
D = 768
F = 1536
E = 16
EPB = 2
TOPK = 4
TBLK = 128


def _moe_body(x_ref, gw_ref, w1_ref, b1_ref, w2_ref, b2_ref, out_ref,
              probs_ref):
    g = pl.program_id(0)

    @pl.when(g == 0)
    def _gating():
        xf = x_ref[...]
        logits = jax.lax.dot_general(
            xf, gw_ref[...], (((1,), (1,)), ((), ())),
            preferred_element_type=jnp.float32)
        m = jnp.max(logits, axis=1, keepdims=True)
        p = jnp.exp(logits - m)
        p = p / jnp.sum(p, axis=1, keepdims=True)
        lane = jax.lax.broadcasted_iota(jnp.int32, (TOKENS, E), 1)
        work = p
        sel = jnp.zeros((TOKENS, E), jnp.float32)
        for _ in range(TOPK):
            mx = jnp.max(work, axis=1, keepdims=True)
            cand = jnp.where(work == mx, lane, E)
            first = jnp.min(cand, axis=1, keepdims=True)
            onehot = lane == first
            sel = jnp.where(onehot, 1.0, sel)
            work = jnp.where(onehot, -1.0, work)
        probs_ref[...] = p * sel
        out_ref[...] = xf

    lane = jax.lax.broadcasted_iota(jnp.int32, (TOKENS, E), 1)
    wcols = []
    ws = []
    for k in range(EPB):
        e = g * EPB + k
        wcols.append(jnp.sum(probs_ref[...] * jnp.where(lane == e, 1.0, 0.0),
                             axis=1, keepdims=True))
        ws.append((w1_ref[k].astype(jnp.bfloat16),
                   w2_ref[k].astype(jnp.bfloat16),
                   b1_ref[k], b2_ref[k]))
    for j in range(TOKENS // TBLK):
        xb = x_ref[pl.ds(j * TBLK, TBLK), :].astype(jnp.bfloat16)
        acc = None
        for k in range(EPB):
            w1, w2, b1v, b2v = ws[k]
            h = jax.lax.dot_general(xb, w1, (((1,), (1,)), ((), ())),
                                    preferred_element_type=jnp.float32)
            h = jnp.maximum((h + b1v).astype(jnp.bfloat16), 0)
            y = jax.lax.dot_general(h, w2, (((1,), (1,)), ((), ())),
                                    preferred_element_type=jnp.float32)
            y = y + b2v
            wj = jax.lax.slice(wcols[k], (j * TBLK, 0), ((j + 1) * TBLK, 1))
            acc = wj * y if acc is None else acc + wj * y
        out_ref[pl.ds(j * TBLK, TBLK), :] += acc


def kernel(x, gate_w, W1, b1, W2, b2):
    return pl.pallas_call(
        _moe_body,
        grid=(E // EPB,),
        in_specs=[
            pl.BlockSpec((TOKENS, D), lambda g: (0, 0)),
            pl.BlockSpec((E, D), lambda g: (0, 0)),
            pl.BlockSpec((EPB, F, D), lambda g: (g, 0, 0)),
            pl.BlockSpec((EPB, 1, F), lambda g: (g, 0, 0)),
            pl.BlockSpec((EPB, D, F), lambda g: (g, 0, 0)),
            pl.BlockSpec((EPB, 1, D), lambda g: (g, 0, 0)),
        ],
        out_specs=pl.BlockSpec((TOKENS, D), lambda g: (0, 0)),
        out_shape=jax.ShapeDtypeStruct((TOKENS, D), jnp.float32),
        scratch_shapes=[pltpu.VMEM((TOKENS, E), jnp.float32)],
        compiler_params=pltpu.CompilerParams(
            vmem_limit_bytes=100 * 1024 * 1024),
    )(x, gate_w, W1, b1.reshape(E, 1, F), W2, b2.reshape(E, 1, D))


# speedup vs baseline: 2.1925x; 2.1925x over previous
import jax
import jax.numpy as jnp
from jax.experimental import pallas as pl
from jax.experimental.pallas import tpu as pltpu

TOKENS = 2048
D = 768
F = 1536
E = 16
TOPK = 4
TBLK = 1024


def _moe_body(x_ref, gw_ref, w1_ref, b1_ref, w2_ref, b2a_ref, out_ref,
              probs_ref, xbf_ref):
    e = pl.program_id(0)

    @pl.when(e == 0)
    def _gating():
        xf = x_ref[...]
        logits = jax.lax.dot_general(
            xf, gw_ref[...], (((1,), (1,)), ((), ())),
            preferred_element_type=jnp.float32)
        m = jnp.max(logits, axis=1, keepdims=True)
        p = jnp.exp(logits - m)
        p = p / jnp.sum(p, axis=1, keepdims=True)
        lane = jax.lax.broadcasted_iota(jnp.int32, (TOKENS, E), 1)
        work = p
        sel = jnp.zeros((TOKENS, E), jnp.float32)
        for _ in range(TOPK):
            mx = jnp.max(work, axis=1, keepdims=True)
            cand = jnp.where(work == mx, lane, E)
            first = jnp.min(cand, axis=1, keepdims=True)
            onehot = lane == first
            sel = jnp.where(onehot, 1.0, sel)
            work = jnp.where(onehot, -1.0, work)
        probs = p * sel
        probs_ref[...] = probs
        out_ref[...] = xf + jax.lax.dot_general(
            probs, b2a_ref[...], (((1,), (0,)), ((), ())),
            preferred_element_type=jnp.float32)
        xbf_ref[...] = xf.astype(jnp.bfloat16)

    lane = jax.lax.broadcasted_iota(jnp.int32, (TOKENS, E), 1)
    wcol = jnp.sum(probs_ref[...] * jnp.where(lane == e, 1.0, 0.0),
                   axis=1, keepdims=True)
    w1 = w1_ref[0].astype(jnp.bfloat16)
    w2 = w2_ref[0].astype(jnp.bfloat16)
    b1v = b1_ref[0]
    for j in range(TOKENS // TBLK):
        xb = xbf_ref[pl.ds(j * TBLK, TBLK), :]
        h = jax.lax.dot_general(xb, w1, (((1,), (1,)), ((), ())),
                                preferred_element_type=jnp.float32)
        h = jnp.maximum((h + b1v).astype(jnp.bfloat16), 0)
        y = jax.lax.dot_general(h, w2, (((1,), (1,)), ((), ())),
                                preferred_element_type=jnp.float32)
        wj = jax.lax.slice(wcol, (j * TBLK, 0), ((j + 1) * TBLK, 1))
        out_ref[pl.ds(j * TBLK, TBLK), :] += wj * y


def kernel(x, gate_w, W1, b1, W2, b2):
    return pl.pallas_call(
        _moe_body,
        grid=(E,),
        in_specs=[
            pl.BlockSpec((TOKENS, D), lambda e: (0, 0)),
            pl.BlockSpec((E, D), lambda e: (0, 0)),
            pl.BlockSpec((1, F, D), lambda e: (e, 0, 0)),
            pl.BlockSpec((1, 1, F), lambda e: (e, 0, 0)),
            pl.BlockSpec((1, D, F), lambda e: (e, 0, 0)),
            pl.BlockSpec((E, D), lambda e: (0, 0)),
        ],
        out_specs=pl.BlockSpec((TOKENS, D), lambda e: (0, 0)),
        out_shape=jax.ShapeDtypeStruct((TOKENS, D), jnp.float32),
        scratch_shapes=[pltpu.VMEM((TOKENS, E), jnp.float32),
                        pltpu.VMEM((TOKENS, D), jnp.bfloat16)],
    )(x, gate_w, W1, b1.reshape(E, 1, F), W2, b2)
